# baseline (speedup 1.0000x reference)
import functools

import jax
import jax.numpy as jnp
from jax.experimental import pallas as pl
from jax.experimental.pallas import tpu as pltpu
from jax.experimental.pallas import tpu_sc as plsc

B = 1024
D = 128
C = 2000
CP = 2048
K = 10
BB = 256
NBLK = B // BB
NCH = CP // 128

NTILES = 16
SPB = B // NTILES
RPT = CP // NTILES

_DN_T = (((1,), (1,)), ((), ()))
_DN_ROW = (((1,), (0,)), ((), ()))
_BIG = jnp.iinfo(jnp.int32).max


def _fold_lanes(x, op):
    m = x[:, 0:128]
    for k in range(1, NCH):
        m = op(m, x[:, k * 128:(k + 1) * 128])
    return m


def _min_lanes(x):
    return jnp.min(_fold_lanes(x, jnp.minimum), axis=1, keepdims=True)


def _max_lanes(x):
    return jnp.max(_fold_lanes(x, jnp.maximum), axis=1, keepdims=True)


def _first_index_where(cond, iota):
    cand = None
    for k in range(NCH):
        sl = slice(k * 128, (k + 1) * 128)
        c = jnp.where(cond[:, sl], iota[:, sl], _BIG)
        cand = c if cand is None else jnp.minimum(cand, c)
    return jnp.min(cand, axis=1, keepdims=True)


def _act_body(data_ref, labels_ref, n_ref, mu_ref, s_ref, cl_ref,
              scores_ref, pred_ref, clusters_ref, j_ref, nnew_ref, winv_ref,
              iv_ref, muiv_ref, t3_ref, assign_ref, claug_ref, count_ref):
    i = pl.program_id(0)

    @pl.when(i == 0)
    def _init():
        var = s_ref[:] / jnp.maximum(n_ref[:], 1.0)[:, None] + 1e-6
        iv = 1.0 / var
        iv_ref[:] = iv
        muiv_ref[:] = (2.0 * mu_ref[:]) * iv
        t3_ref[:] = jnp.sum(mu_ref[:] * mu_ref[:] * iv, axis=1)[None, :]
        cl = cl_ref[:]
        cidx = jax.lax.broadcasted_iota(jnp.int32, cl.shape, 1)
        assign_ref[:] = jnp.sum(cl * cidx, axis=1)[None, :]
        claug_ref[:] = cl.astype(jnp.float32)
        count_ref[:] = jnp.zeros_like(count_ref)

    @pl.when(i < NBLK)
    def _activation():
        b = i
        x = data_ref[:]
        t1 = jax.lax.dot_general(x * x, iv_ref[:], _DN_T,
                                 preferred_element_type=jnp.float32)
        t2 = jax.lax.dot_general(x, muiv_ref[:], _DN_T,
                                 preferred_element_type=jnp.float32)
        d2 = jnp.maximum(t1 - t2 + t3_ref[:], 0.0)
        dmin = _min_lanes(d2)
        g = jnp.exp(-0.5 * (d2 - dmin))

        iota = jax.lax.broadcasted_iota(jnp.int32, (BB, CP), 1)
        cc = _first_index_where(g == 1.0, iota)
        gm = jnp.where(labels_ref[:] == assign_ref[:], g, 0.0)
        mg = _max_lanes(gm)
        jc = _first_index_where(gm == mg, iota)

        s = jnp.sum(_fold_lanes(g, jnp.add), axis=1, keepdims=True)
        gn = g / (s + 1e-12)
        scores = jax.lax.dot_general(gn, claug_ref[:], _DN_ROW,
                                     preferred_element_type=jnp.float32)
        m = jnp.max(scores, axis=1, keepdims=True)
        kidx = jax.lax.broadcasted_iota(jnp.int32, scores.shape, 1)
        pc = jnp.min(jnp.where(scores == m, kidx, _BIG), axis=1, keepdims=True)

        onehot = (jc == iota).astype(jnp.bfloat16)
        count_ref[:] += jax.lax.dot_general(
            jnp.ones((1, BB), jnp.bfloat16), onehot, _DN_ROW,
            preferred_element_type=jnp.float32)
        j_ref[pl.ds(b * BB, BB)] = jc[:, 0]
        scores_ref[pl.ds(b * BB, BB), :] = scores
        pred_ref[pl.ds(b * BB, BB)] = pc[:, 0]
        clusters_ref[pl.ds(b * BB, BB)] = cc[:, 0]

    @pl.when(i == NBLK)
    def _tail():
        nn = n_ref[:] + count_ref[0, :]
        nnew_ref[:] = nn
        winv_ref[:] = jnp.broadcast_to((1.0 / nn)[:, None], (CP, D))


def _activation_call(data, labels_col, n_p, mu_p, s_p, cl_p):
    out_shapes = (
        jax.ShapeDtypeStruct((B, K), jnp.float32),
        jax.ShapeDtypeStruct((B,), jnp.int32),
        jax.ShapeDtypeStruct((B,), jnp.int32),
        jax.ShapeDtypeStruct((B,), jnp.int32),
        jax.ShapeDtypeStruct((CP,), jnp.float32),
        jax.ShapeDtypeStruct((CP, D), jnp.float32),
    )
    blk = lambda i: (jnp.minimum(i, NBLK - 1), 0)
    in_specs = [
        pl.BlockSpec((BB, D), blk),
        pl.BlockSpec((BB, 1), blk),
        pl.BlockSpec((CP,), lambda i: (0,)),
        pl.BlockSpec((CP, D), lambda i: (0, 0)),
        pl.BlockSpec((CP, D), lambda i: (0, 0)),
        pl.BlockSpec((CP, K), lambda i: (0, 0)),
    ]
    out_specs = (
        pl.BlockSpec((B, K), lambda i: (0, 0)),
        pl.BlockSpec((B,), lambda i: (0,)),
        pl.BlockSpec((B,), lambda i: (0,)),
        pl.BlockSpec((B,), lambda i: (0,)),
        pl.BlockSpec((CP,), lambda i: (0,)),
        pl.BlockSpec((CP, D), lambda i: (0, 0)),
    )
    scratch = [
        pltpu.VMEM((CP, D), jnp.float32),
        pltpu.VMEM((CP, D), jnp.float32),
        pltpu.VMEM((1, CP), jnp.float32),
        pltpu.VMEM((1, CP), jnp.int32),
        pltpu.VMEM((CP, K), jnp.float32),
        pltpu.VMEM((1, CP), jnp.float32),
    ]
    return pl.pallas_call(
        _act_body, grid=(NBLK + 1,), in_specs=in_specs, out_specs=out_specs,
        out_shape=out_shapes, scratch_shapes=scratch,
    )(data, labels_col, n_p, mu_p, s_p, cl_p)


def _sc_update_body(data_hbm, j_hbm, winv_hbm, mu_hbm, s_hbm,
                    munew_hbm, snew_hbm,
                    idx_v, x_v, row_v, upd_v, w_v, sem, table_sh):
    cid = jax.lax.axis_index("c")
    sid = jax.lax.axis_index("s")
    tsl = pl.ds(sid * RPT, RPT)

    @pl.when(cid == 0)
    def _():
        pltpu.sync_copy(mu_hbm.at[tsl], table_sh.at[tsl])

    @pl.when(cid == 1)
    def _():
        pltpu.sync_copy(s_hbm.at[tsl], table_sh.at[tsl])

    bsl = pl.ds(sid * SPB, SPB)
    pltpu.sync_copy(j_hbm.at[bsl], idx_v)
    pltpu.sync_copy(data_hbm.at[bsl], x_v)
    pltpu.async_copy(mu_hbm.at[idx_v], row_v, sem).wait()

    @pl.when(cid == 0)
    def _():
        pltpu.async_copy(winv_hbm.at[idx_v], w_v, sem).wait()

    @pl.when(cid == 0)
    def _():
        def rbody(r, carry):
            w16 = w_v[r, pl.ds(0, 16)]
            for k in range(D // 16):
                csl = pl.ds(k * 16, 16)
                u16 = (x_v[r, csl] - row_v[r, csl]) * w16
                upd_v[r, csl] = u16
            return carry

        jax.lax.fori_loop(0, SPB, rbody, 0)

    @pl.when(cid == 1)
    def _():
        def rbody(r, carry):
            for k in range(D // 16):
                csl = pl.ds(k * 16, 16)
                e16 = x_v[r, csl] - row_v[r, csl]
                upd_v[r, csl] = e16 * e16
            return carry

        jax.lax.fori_loop(0, SPB, rbody, 0)

    plsc.subcore_barrier()
    pltpu.sync_copy(upd_v, table_sh.at[idx_v], add=True)
    plsc.subcore_barrier()

    @pl.when(cid == 0)
    def _():
        pltpu.sync_copy(table_sh.at[tsl], munew_hbm.at[tsl])

    @pl.when(cid == 1)
    def _():
        pltpu.sync_copy(table_sh.at[tsl], snew_hbm.at[tsl])


def _update_call(data, j, winv, mu_p, s_p):
    mesh = plsc.VectorSubcoreMesh(core_axis_name="c", subcore_axis_name="s")
    kern = pl.kernel(
        _sc_update_body,
        out_type=[
            jax.ShapeDtypeStruct((CP, D), jnp.float32),
            jax.ShapeDtypeStruct((CP, D), jnp.float32),
        ],
        mesh=mesh,
        scratch_types=[
            pltpu.VMEM((SPB,), jnp.int32),
            pltpu.VMEM((SPB, D), jnp.float32),
            pltpu.VMEM((SPB, D), jnp.float32),
            pltpu.VMEM((SPB, D), jnp.float32),
            pltpu.VMEM((SPB, D), jnp.float32),
            pltpu.SemaphoreType.DMA,
            pltpu.VMEM_SHARED((CP, D), jnp.float32),
        ],
    )
    return kern(data, j, winv, mu_p, s_p)


def kernel(data, labels, n, mu, S_diag, cluster_labels):
    pad = CP - C
    mu_p = jnp.pad(mu, ((0, pad), (0, 0)))
    s_p = jnp.pad(S_diag, ((0, pad), (0, 0)))
    n_p = jnp.pad(n, (0, pad), constant_values=1.0)
    cl_p = jnp.pad(cluster_labels, ((0, pad), (0, 0)))
    labels_col = labels[:, None]

    scores, pred, clusters, j, n_new, winv = _activation_call(
        data, labels_col, n_p, mu_p, s_p, cl_p)
    mu_new, S_new = _update_call(data, j, winv, mu_p, s_p)
    return (scores, pred, clusters, n_new[:C], mu_new[:C], S_new[:C])


# speedup vs baseline: 1.0404x; 1.0404x over previous
import functools

import jax
import jax.numpy as jnp
from jax.experimental import pallas as pl
from jax.experimental.pallas import tpu as pltpu
from jax.experimental.pallas import tpu_sc as plsc

B = 1024
D = 128
C = 2000
CP = 2048
K = 10
BB = 256
NBLK = B // BB
NCH = CP // 128

NTILES = 16
SPB = B // NTILES
RPT = CP // NTILES

_DN_T = (((1,), (1,)), ((), ()))
_DN_ROW = (((1,), (0,)), ((), ()))
_BIG = jnp.iinfo(jnp.int32).max


def _fold_lanes(x, op):
    m = x[:, 0:128]
    for k in range(1, NCH):
        m = op(m, x[:, k * 128:(k + 1) * 128])
    return m


def _min_lanes(x):
    return jnp.min(_fold_lanes(x, jnp.minimum), axis=1, keepdims=True)


def _max_lanes(x):
    return jnp.max(_fold_lanes(x, jnp.maximum), axis=1, keepdims=True)


def _first_index_where(cond, iota):
    cand = None
    for k in range(NCH):
        sl = slice(k * 128, (k + 1) * 128)
        c = jnp.where(cond[:, sl], iota[:, sl], _BIG)
        cand = c if cand is None else jnp.minimum(cand, c)
    return jnp.min(cand, axis=1, keepdims=True)


def _act_body(data_ref, labels_ref, n_ref, mu_ref, s_ref, cl_ref,
              scores_ref, pred_ref, clusters_ref, j_ref, nnew_ref, winv_ref,
              iv_ref, muiv_ref, t3_ref, assign_ref, claug_ref, count_ref):
    i = pl.program_id(0)

    @pl.when(i == 0)
    def _init():
        var = s_ref[:] / jnp.maximum(n_ref[:], 1.0)[:, None] + 1e-6
        iv = 1.0 / var
        iv_ref[:] = iv
        muiv_ref[:] = (2.0 * mu_ref[:]) * iv
        t3_ref[:] = jnp.sum(mu_ref[:] * mu_ref[:] * iv, axis=1)[None, :]
        cl = cl_ref[:]
        cidx = jax.lax.broadcasted_iota(jnp.int32, cl.shape, 1)
        assign_ref[:] = jnp.sum(cl * cidx, axis=1)[None, :]
        claug_ref[:] = cl.astype(jnp.float32)
        count_ref[:] = jnp.zeros_like(count_ref)

    @pl.when(i < NBLK)
    def _activation():
        b = i
        x = data_ref[:]
        t1 = jax.lax.dot_general(x * x, iv_ref[:], _DN_T,
                                 preferred_element_type=jnp.float32)
        t2 = jax.lax.dot_general(x, muiv_ref[:], _DN_T,
                                 preferred_element_type=jnp.float32)
        d2 = jnp.maximum(t1 - t2 + t3_ref[:], 0.0)
        dmin = _min_lanes(d2)
        g = jnp.exp(-0.5 * (d2 - dmin))

        iota = jax.lax.broadcasted_iota(jnp.int32, (BB, CP), 1)
        cc = _first_index_where(g == 1.0, iota)
        gm = jnp.where(labels_ref[:] == assign_ref[:], g, 0.0)
        mg = _max_lanes(gm)
        jc = _first_index_where(gm == mg, iota)

        s = jnp.sum(_fold_lanes(g, jnp.add), axis=1, keepdims=True)
        gn = g / (s + 1e-12)
        scores = jax.lax.dot_general(gn, claug_ref[:], _DN_ROW,
                                     preferred_element_type=jnp.float32)
        m = jnp.max(scores, axis=1, keepdims=True)
        kidx = jax.lax.broadcasted_iota(jnp.int32, scores.shape, 1)
        pc = jnp.min(jnp.where(scores == m, kidx, _BIG), axis=1, keepdims=True)

        onehot = (jc == iota).astype(jnp.bfloat16)
        count_ref[:] += jax.lax.dot_general(
            jnp.ones((1, BB), jnp.bfloat16), onehot, _DN_ROW,
            preferred_element_type=jnp.float32)
        j_ref[pl.ds(b * BB, BB)] = jc[:, 0]
        scores_ref[pl.ds(b * BB, BB), :] = scores
        pred_ref[pl.ds(b * BB, BB)] = pc[:, 0]
        clusters_ref[pl.ds(b * BB, BB)] = cc[:, 0]

    @pl.when(i == NBLK)
    def _tail():
        nn = n_ref[:] + count_ref[0, :]
        nnew_ref[:] = nn
        winv_ref[:] = jnp.broadcast_to((1.0 / nn)[:, None], (CP, D))


def _activation_call(data, labels_col, n_p, mu_p, s_p, cl_p):
    out_shapes = (
        jax.ShapeDtypeStruct((B, K), jnp.float32),
        jax.ShapeDtypeStruct((B,), jnp.int32),
        jax.ShapeDtypeStruct((B,), jnp.int32),
        jax.ShapeDtypeStruct((B,), jnp.int32),
        jax.ShapeDtypeStruct((CP,), jnp.float32),
        jax.ShapeDtypeStruct((CP, D), jnp.float32),
    )
    blk = lambda i: (jnp.minimum(i, NBLK - 1), 0)
    in_specs = [
        pl.BlockSpec((BB, D), blk),
        pl.BlockSpec((BB, 1), blk),
        pl.BlockSpec((CP,), lambda i: (0,)),
        pl.BlockSpec((CP, D), lambda i: (0, 0)),
        pl.BlockSpec((CP, D), lambda i: (0, 0)),
        pl.BlockSpec((CP, K), lambda i: (0, 0)),
    ]
    out_specs = (
        pl.BlockSpec((B, K), lambda i: (0, 0)),
        pl.BlockSpec((B,), lambda i: (0,)),
        pl.BlockSpec((B,), lambda i: (0,)),
        pl.BlockSpec((B,), lambda i: (0,)),
        pl.BlockSpec((CP,), lambda i: (0,)),
        pl.BlockSpec((CP, D), lambda i: (0, 0)),
    )
    scratch = [
        pltpu.VMEM((CP, D), jnp.float32),
        pltpu.VMEM((CP, D), jnp.float32),
        pltpu.VMEM((1, CP), jnp.float32),
        pltpu.VMEM((1, CP), jnp.int32),
        pltpu.VMEM((CP, K), jnp.float32),
        pltpu.VMEM((1, CP), jnp.float32),
    ]
    return pl.pallas_call(
        _act_body, grid=(NBLK + 1,), in_specs=in_specs, out_specs=out_specs,
        out_shape=out_shapes, scratch_shapes=scratch,
    )(data, labels_col, n_p, mu_p, s_p, cl_p)


def _sc_update_body(data_hbm, j_hbm, winv_hbm, mu_hbm, s_hbm,
                    munew_hbm, snew_hbm,
                    idx_v, x_v, row_v, upd_v, w_v,
                    sem_t, sem_d, sem_r, sem_w, table_sh):
    cid = jax.lax.axis_index("c")
    sid = jax.lax.axis_index("s")
    tsl = pl.ds(sid * RPT, RPT)
    bsl = pl.ds(sid * SPB, SPB)

    pltpu.sync_copy(j_hbm.at[bsl], idx_v)

    @pl.when(cid == 0)
    def _():
        tcopy = pltpu.async_copy(mu_hbm.at[tsl], table_sh.at[tsl], sem_t)
        dcopy = pltpu.async_copy(data_hbm.at[bsl], x_v, sem_d)
        rcopy = pltpu.async_copy(mu_hbm.at[idx_v], row_v, sem_r)
        wcopy = pltpu.async_copy(winv_hbm.at[idx_v], w_v, sem_w)
        dcopy.wait()
        rcopy.wait()
        wcopy.wait()

        def rbody(r, carry):
            w16 = w_v[r, pl.ds(0, 16)]
            for k in range(D // 16):
                csl = pl.ds(k * 16, 16)
                upd_v[r, csl] = (x_v[r, csl] - row_v[r, csl]) * w16
            return carry

        jax.lax.fori_loop(0, SPB, rbody, 0)
        tcopy.wait()

    @pl.when(cid == 1)
    def _():
        tcopy = pltpu.async_copy(s_hbm.at[tsl], table_sh.at[tsl], sem_t)
        dcopy = pltpu.async_copy(data_hbm.at[bsl], x_v, sem_d)
        rcopy = pltpu.async_copy(mu_hbm.at[idx_v], row_v, sem_r)
        dcopy.wait()
        rcopy.wait()

        def rbody(r, carry):
            for k in range(D // 16):
                csl = pl.ds(k * 16, 16)
                e16 = x_v[r, csl] - row_v[r, csl]
                upd_v[r, csl] = e16 * e16
            return carry

        jax.lax.fori_loop(0, SPB, rbody, 0)
        tcopy.wait()

    plsc.subcore_barrier()
    pltpu.sync_copy(upd_v, table_sh.at[idx_v], add=True)
    plsc.subcore_barrier()

    @pl.when(cid == 0)
    def _():
        pltpu.sync_copy(table_sh.at[tsl], munew_hbm.at[tsl])

    @pl.when(cid == 1)
    def _():
        pltpu.sync_copy(table_sh.at[tsl], snew_hbm.at[tsl])


def _update_call(data, j, winv, mu_p, s_p):
    mesh = plsc.VectorSubcoreMesh(core_axis_name="c", subcore_axis_name="s")
    kern = pl.kernel(
        _sc_update_body,
        out_type=[
            jax.ShapeDtypeStruct((CP, D), jnp.float32),
            jax.ShapeDtypeStruct((CP, D), jnp.float32),
        ],
        mesh=mesh,
        scratch_types=[
            pltpu.VMEM((SPB,), jnp.int32),
            pltpu.VMEM((SPB, D), jnp.float32),
            pltpu.VMEM((SPB, D), jnp.float32),
            pltpu.VMEM((SPB, D), jnp.float32),
            pltpu.VMEM((SPB, D), jnp.float32),
            pltpu.SemaphoreType.DMA,
            pltpu.SemaphoreType.DMA,
            pltpu.SemaphoreType.DMA,
            pltpu.SemaphoreType.DMA,
            pltpu.VMEM_SHARED((CP, D), jnp.float32),
        ],
    )
    return kern(data, j, winv, mu_p, s_p)


def kernel(data, labels, n, mu, S_diag, cluster_labels):
    pad = CP - C
    mu_p = jnp.pad(mu, ((0, pad), (0, 0)))
    s_p = jnp.pad(S_diag, ((0, pad), (0, 0)))
    n_p = jnp.pad(n, (0, pad), constant_values=1.0)
    cl_p = jnp.pad(cluster_labels, ((0, pad), (0, 0)))
    labels_col = labels[:, None]

    scores, pred, clusters, j, n_new, winv = _activation_call(
        data, labels_col, n_p, mu_p, s_p, cl_p)
    mu_new, S_new = _update_call(data, j, winv, mu_p, s_p)
    return (scores, pred, clusters, n_new[:C], mu_new[:C], S_new[:C])


# speedup vs baseline: 1.2276x; 1.1799x over previous
import jax
import jax.numpy as jnp
from jax.experimental import pallas as pl
from jax.experimental.pallas import tpu as pltpu
from jax.experimental.pallas import tpu_sc as plsc

B = 1024
D = 128
C = 2000
CP = 2048
K = 10
BB = 256
NBLK = B // BB
NCH = CP // 128

NTILES = 16
SPB = B // NTILES
RPT = C // NTILES

_DN_T = (((1,), (1,)), ((), ()))
_DN_ROW = (((1,), (0,)), ((), ()))
_BIG = jnp.iinfo(jnp.int32).max


def _fold_lanes(x, op):
    m = x[:, 0:128]
    for k in range(1, NCH):
        m = op(m, x[:, k * 128:(k + 1) * 128])
    return m


def _min_lanes(x):
    return jnp.min(_fold_lanes(x, jnp.minimum), axis=1, keepdims=True)


def _max_lanes(x):
    return jnp.max(_fold_lanes(x, jnp.maximum), axis=1, keepdims=True)


def _first_index_where(cond, iota):
    cand = None
    for k in range(NCH):
        sl = slice(k * 128, (k + 1) * 128)
        c = jnp.where(cond[:, sl], iota[:, sl], _BIG)
        cand = c if cand is None else jnp.minimum(cand, c)
    return jnp.min(cand, axis=1, keepdims=True)


def _act_body(data_ref, labels_ref, n_ref, mu_ref, s_ref, cl_ref,
              scores_ref, pred_ref, clusters_ref, j_ref, nnew_ref, winv_ref,
              iv_ref, muiv_ref, t3_ref, assign_ref, claug_ref, count_ref):
    i = pl.program_id(0)

    @pl.when(i == 0)
    def _init():
        var = s_ref[:] / jnp.maximum(n_ref[:], 1.0)[:, None] + 1e-6
        iv = 1.0 / var
        iv_ref[pl.ds(0, C), :] = iv
        iv_ref[pl.ds(C, CP - C), :] = jnp.full((CP - C, D), 1e6, jnp.float32)
        muiv_ref[pl.ds(0, C), :] = (2.0 * mu_ref[:]) * iv
        muiv_ref[pl.ds(C, CP - C), :] = jnp.zeros((CP - C, D), jnp.float32)
        t3_ref[:, pl.ds(0, C)] = jnp.sum(
            mu_ref[:] * mu_ref[:] * iv, axis=1)[None, :]
        t3_ref[:, pl.ds(C, CP - C)] = jnp.zeros((1, CP - C), jnp.float32)
        claug_ref[pl.ds(0, C), :] = cl_ref[:].astype(jnp.float32)
        claug_ref[pl.ds(C, CP - C), :] = jnp.zeros((CP - C, K), jnp.float32)
        cidx = jax.lax.broadcasted_iota(
            jnp.int32, (CP, K), 1).astype(jnp.float32)
        assign_ref[:] = jnp.sum(claug_ref[:] * cidx, axis=1)[None, :]
        assign_ref[:, pl.ds(C, CP - C)] = jnp.full((1, CP - C), -1.0,
                                                   jnp.float32)
        count_ref[:] = jnp.zeros_like(count_ref)

    @pl.when(i < NBLK)
    def _activation():
        b = i
        x = data_ref[:]
        t1 = jax.lax.dot_general(x * x, iv_ref[:], _DN_T,
                                 preferred_element_type=jnp.float32)
        t2 = jax.lax.dot_general(x, muiv_ref[:], _DN_T,
                                 preferred_element_type=jnp.float32)
        d2 = jnp.maximum(t1 - t2 + t3_ref[:], 0.0)
        dmin = _min_lanes(d2)
        g = jnp.exp(-0.5 * (d2 - dmin))

        iota = jax.lax.broadcasted_iota(jnp.int32, (BB, CP), 1)
        cc = _first_index_where(g == 1.0, iota)
        lab = labels_ref[:].astype(jnp.float32)
        gm = jnp.where(lab == assign_ref[:], g, 0.0)
        mg = _max_lanes(gm)
        jc = _first_index_where(gm == mg, iota)

        s = jnp.sum(_fold_lanes(g, jnp.add), axis=1, keepdims=True)
        gn = g / (s + 1e-12)
        scores = jax.lax.dot_general(gn, claug_ref[:], _DN_ROW,
                                     preferred_element_type=jnp.float32)
        m = jnp.max(scores, axis=1, keepdims=True)
        kidx = jax.lax.broadcasted_iota(jnp.int32, scores.shape, 1)
        pc = jnp.min(jnp.where(scores == m, kidx, _BIG), axis=1, keepdims=True)

        onehot = (jc == iota).astype(jnp.bfloat16)
        count_ref[:] += jax.lax.dot_general(
            jnp.ones((1, BB), jnp.bfloat16), onehot, _DN_ROW,
            preferred_element_type=jnp.float32)
        j_ref[pl.ds(b * BB, BB)] = jc[:, 0]
        scores_ref[pl.ds(b * BB, BB), :] = scores
        pred_ref[pl.ds(b * BB, BB)] = pc[:, 0]
        clusters_ref[pl.ds(b * BB, BB)] = cc[:, 0]

    @pl.when(i == NBLK)
    def _tail():
        nn = n_ref[:] + count_ref[0, pl.ds(0, C)]
        nnew_ref[:] = nn
        winv_ref[:] = jnp.broadcast_to((1.0 / nn)[:, None], (C, D))


def _activation_call(data, labels_col, n, mu, S_diag, cl):
    out_shapes = (
        jax.ShapeDtypeStruct((B, K), jnp.float32),
        jax.ShapeDtypeStruct((B,), jnp.int32),
        jax.ShapeDtypeStruct((B,), jnp.int32),
        jax.ShapeDtypeStruct((B,), jnp.int32),
        jax.ShapeDtypeStruct((C,), jnp.float32),
        jax.ShapeDtypeStruct((C, D), jnp.float32),
    )
    blk = lambda i: (jnp.minimum(i, NBLK - 1), 0)
    in_specs = [
        pl.BlockSpec((BB, D), blk),
        pl.BlockSpec((BB, 1), blk),
        pl.BlockSpec((C,), lambda i: (0,)),
        pl.BlockSpec((C, D), lambda i: (0, 0)),
        pl.BlockSpec((C, D), lambda i: (0, 0)),
        pl.BlockSpec((C, K), lambda i: (0, 0)),
    ]
    out_specs = (
        pl.BlockSpec((B, K), lambda i: (0, 0)),
        pl.BlockSpec((B,), lambda i: (0,)),
        pl.BlockSpec((B,), lambda i: (0,)),
        pl.BlockSpec((B,), lambda i: (0,)),
        pl.BlockSpec((C,), lambda i: (0,)),
        pl.BlockSpec((C, D), lambda i: (0, 0)),
    )
    scratch = [
        pltpu.VMEM((CP, D), jnp.float32),
        pltpu.VMEM((CP, D), jnp.float32),
        pltpu.VMEM((1, CP), jnp.float32),
        pltpu.VMEM((1, CP), jnp.float32),
        pltpu.VMEM((CP, K), jnp.float32),
        pltpu.VMEM((1, CP), jnp.float32),
    ]
    return pl.pallas_call(
        _act_body, grid=(NBLK + 1,), in_specs=in_specs, out_specs=out_specs,
        out_shape=out_shapes, scratch_shapes=scratch,
    )(data, labels_col, n, mu, S_diag, cl)


def _sc_update_body(data_hbm, j_hbm, winv_hbm, mu_hbm, s_hbm,
                    munew_hbm, snew_hbm,
                    idx_v, x_v, row_v, upd_v, w_v,
                    sem_t, sem_d, sem_r, sem_w, table_sh):
    cid = jax.lax.axis_index("c")
    sid = jax.lax.axis_index("s")
    tsl = pl.ds(jnp.minimum(sid * 128, C - 128), 128)
    bsl = pl.ds(sid * SPB, SPB)

    pltpu.sync_copy(j_hbm.at[bsl], idx_v)

    @pl.when(cid == 0)
    def _():
        tcopy = pltpu.async_copy(mu_hbm.at[tsl], table_sh.at[tsl], sem_t)
        dcopy = pltpu.async_copy(data_hbm.at[bsl], x_v, sem_d)
        rcopy = pltpu.async_copy(mu_hbm.at[idx_v], row_v, sem_r)
        wcopy = pltpu.async_copy(winv_hbm.at[idx_v], w_v, sem_w)
        dcopy.wait()
        rcopy.wait()
        wcopy.wait()

        def rbody(r, carry):
            w16 = w_v[r, pl.ds(0, 16)]
            for k in range(D // 16):
                csl = pl.ds(k * 16, 16)
                upd_v[r, csl] = (x_v[r, csl] - row_v[r, csl]) * w16
            return carry

        jax.lax.fori_loop(0, SPB, rbody, 0)
        tcopy.wait()

    @pl.when(cid == 1)
    def _():
        tcopy = pltpu.async_copy(s_hbm.at[tsl], table_sh.at[tsl], sem_t)
        dcopy = pltpu.async_copy(data_hbm.at[bsl], x_v, sem_d)
        rcopy = pltpu.async_copy(mu_hbm.at[idx_v], row_v, sem_r)
        dcopy.wait()
        rcopy.wait()

        def rbody(r, carry):
            for k in range(D // 16):
                csl = pl.ds(k * 16, 16)
                e16 = x_v[r, csl] - row_v[r, csl]
                upd_v[r, csl] = e16 * e16
            return carry

        jax.lax.fori_loop(0, SPB, rbody, 0)
        tcopy.wait()

    plsc.subcore_barrier()
    pltpu.sync_copy(upd_v, table_sh.at[idx_v], add=True)
    plsc.subcore_barrier()

    @pl.when(cid == 0)
    def _():
        pltpu.sync_copy(table_sh.at[tsl], munew_hbm.at[tsl])

    @pl.when(cid == 1)
    def _():
        pltpu.sync_copy(table_sh.at[tsl], snew_hbm.at[tsl])


def _update_call(data, j, winv, mu, S_diag):
    mesh = plsc.VectorSubcoreMesh(core_axis_name="c", subcore_axis_name="s")
    kern = pl.kernel(
        _sc_update_body,
        out_type=[
            jax.ShapeDtypeStruct((C, D), jnp.float32),
            jax.ShapeDtypeStruct((C, D), jnp.float32),
        ],
        mesh=mesh,
        scratch_types=[
            pltpu.VMEM((SPB,), jnp.int32),
            pltpu.VMEM((SPB, D), jnp.float32),
            pltpu.VMEM((SPB, D), jnp.float32),
            pltpu.VMEM((SPB, D), jnp.float32),
            pltpu.VMEM((SPB, D), jnp.float32),
            pltpu.SemaphoreType.DMA,
            pltpu.SemaphoreType.DMA,
            pltpu.SemaphoreType.DMA,
            pltpu.SemaphoreType.DMA,
            pltpu.VMEM_SHARED((C, D), jnp.float32),
        ],
    )
    return kern(data, j, winv, mu, S_diag)


def kernel(data, labels, n, mu, S_diag, cluster_labels):
    labels_col = labels[:, None]
    scores, pred, clusters, j, n_new, winv = _activation_call(
        data, labels_col, n, mu, S_diag, cluster_labels)
    mu_new, S_new = _update_call(data, j, winv, mu, S_diag)
    return (scores, pred, clusters, n_new, mu_new, S_new)


# speedup vs baseline: 1.2449x; 1.0141x over previous
import jax
import jax.numpy as jnp
from jax.experimental import pallas as pl
from jax.experimental.pallas import tpu as pltpu
from jax.experimental.pallas import tpu_sc as plsc

B = 1024
D = 128
C = 2000
CP = 2048
K = 10
BB = 512
NBLK = B // BB
NCH = CP // 128

NTILES = 16
SPB = B // NTILES
RPT = C // NTILES

_DN_T = (((1,), (1,)), ((), ()))
_DN_ROW = (((1,), (0,)), ((), ()))
_BIG = jnp.iinfo(jnp.int32).max


def _fold_lanes(x, op):
    m = x[:, 0:128]
    for k in range(1, NCH):
        m = op(m, x[:, k * 128:(k + 1) * 128])
    return m


def _min_lanes(x):
    return jnp.min(_fold_lanes(x, jnp.minimum), axis=1, keepdims=True)


def _max_lanes(x):
    return jnp.max(_fold_lanes(x, jnp.maximum), axis=1, keepdims=True)


def _first_index_where(cond, iota):
    cand = None
    for k in range(NCH):
        sl = slice(k * 128, (k + 1) * 128)
        c = jnp.where(cond[:, sl], iota[:, sl], _BIG)
        cand = c if cand is None else jnp.minimum(cand, c)
    return jnp.min(cand, axis=1, keepdims=True)


def _act_body(data_ref, labels_ref, n_ref, mu_ref, s_ref, cl_ref,
              scores_ref, pred_ref, clusters_ref, j_ref, nnew_ref, winv_ref,
              iv_ref, muiv_ref, t3_ref, assign_ref, claug_ref, count_ref):
    i = pl.program_id(0)

    @pl.when(i == 0)
    def _init():
        var = s_ref[:] / jnp.maximum(n_ref[:], 1.0)[:, None] + 1e-6
        iv = 1.0 / var
        iv_ref[pl.ds(0, C), :] = iv
        iv_ref[pl.ds(C, CP - C), :] = jnp.full((CP - C, D), 1e6, jnp.float32)
        muiv_ref[pl.ds(0, C), :] = (2.0 * mu_ref[:]) * iv
        muiv_ref[pl.ds(C, CP - C), :] = jnp.zeros((CP - C, D), jnp.float32)
        t3_ref[:, pl.ds(0, C)] = jnp.sum(
            mu_ref[:] * mu_ref[:] * iv, axis=1)[None, :]
        t3_ref[:, pl.ds(C, CP - C)] = jnp.zeros((1, CP - C), jnp.float32)
        claug_ref[pl.ds(0, C), :] = cl_ref[:].astype(jnp.float32)
        claug_ref[pl.ds(C, CP - C), :] = jnp.zeros((CP - C, K), jnp.float32)
        cidx = jax.lax.broadcasted_iota(
            jnp.int32, (CP, K), 1).astype(jnp.float32)
        assign_ref[:] = jnp.sum(claug_ref[:] * cidx, axis=1)[None, :]
        assign_ref[:, pl.ds(C, CP - C)] = jnp.full((1, CP - C), -1.0,
                                                   jnp.float32)
        count_ref[:] = jnp.zeros_like(count_ref)

    @pl.when(i < NBLK)
    def _activation():
        b = i
        x = data_ref[:]
        t1 = jax.lax.dot_general(x * x, iv_ref[:], _DN_T,
                                 preferred_element_type=jnp.float32)
        t2 = jax.lax.dot_general(x, muiv_ref[:], _DN_T,
                                 preferred_element_type=jnp.float32)
        d2 = jnp.maximum(t1 - t2 + t3_ref[:], 0.0)
        dmin = _min_lanes(d2)
        g = jnp.exp(-0.5 * (d2 - dmin))

        iota = jax.lax.broadcasted_iota(jnp.int32, (BB, CP), 1)
        cc = _first_index_where(g == 1.0, iota)
        lab = labels_ref[:].astype(jnp.float32)
        gm = jnp.where(lab == assign_ref[:], g, 0.0)
        mg = _max_lanes(gm)
        jc = _first_index_where(gm == mg, iota)

        s = jnp.sum(_fold_lanes(g, jnp.add), axis=1, keepdims=True)
        gn = g / (s + 1e-12)
        scores = jax.lax.dot_general(gn, claug_ref[:], _DN_ROW,
                                     preferred_element_type=jnp.float32)
        m = jnp.max(scores, axis=1, keepdims=True)
        kidx = jax.lax.broadcasted_iota(jnp.int32, scores.shape, 1)
        pc = jnp.min(jnp.where(scores == m, kidx, _BIG), axis=1, keepdims=True)

        onehot = (jc == iota).astype(jnp.bfloat16)
        count_ref[:] += jax.lax.dot_general(
            jnp.ones((1, BB), jnp.bfloat16), onehot, _DN_ROW,
            preferred_element_type=jnp.float32)
        j_ref[pl.ds(b * BB, BB)] = jc[:, 0]
        scores_ref[pl.ds(b * BB, BB), :] = scores
        pred_ref[pl.ds(b * BB, BB)] = pc[:, 0]
        clusters_ref[pl.ds(b * BB, BB)] = cc[:, 0]

    @pl.when(i == NBLK)
    def _tail():
        nn = n_ref[:] + count_ref[0, pl.ds(0, C)]
        nnew_ref[:] = nn
        winv_ref[:] = jnp.broadcast_to((1.0 / nn)[:, None], (C, D))


def _activation_call(data, labels_col, n, mu, S_diag, cl):
    out_shapes = (
        jax.ShapeDtypeStruct((B, K), jnp.float32),
        jax.ShapeDtypeStruct((B,), jnp.int32),
        jax.ShapeDtypeStruct((B,), jnp.int32),
        jax.ShapeDtypeStruct((B,), jnp.int32),
        jax.ShapeDtypeStruct((C,), jnp.float32),
        jax.ShapeDtypeStruct((C, D), jnp.float32),
    )
    blk = lambda i: (jnp.minimum(i, NBLK - 1), 0)
    in_specs = [
        pl.BlockSpec((BB, D), blk),
        pl.BlockSpec((BB, 1), blk),
        pl.BlockSpec((C,), lambda i: (0,)),
        pl.BlockSpec((C, D), lambda i: (0, 0)),
        pl.BlockSpec((C, D), lambda i: (0, 0)),
        pl.BlockSpec((C, K), lambda i: (0, 0)),
    ]
    out_specs = (
        pl.BlockSpec((B, K), lambda i: (0, 0)),
        pl.BlockSpec((B,), lambda i: (0,)),
        pl.BlockSpec((B,), lambda i: (0,)),
        pl.BlockSpec((B,), lambda i: (0,)),
        pl.BlockSpec((C,), lambda i: (0,)),
        pl.BlockSpec((C, D), lambda i: (0, 0)),
    )
    scratch = [
        pltpu.VMEM((CP, D), jnp.float32),
        pltpu.VMEM((CP, D), jnp.float32),
        pltpu.VMEM((1, CP), jnp.float32),
        pltpu.VMEM((1, CP), jnp.float32),
        pltpu.VMEM((CP, K), jnp.float32),
        pltpu.VMEM((1, CP), jnp.float32),
    ]
    return pl.pallas_call(
        _act_body, grid=(NBLK + 1,), in_specs=in_specs, out_specs=out_specs,
        out_shape=out_shapes, scratch_shapes=scratch,
    )(data, labels_col, n, mu, S_diag, cl)


def _sc_update_body(data_hbm, j_hbm, winv_hbm, mu_hbm, s_hbm,
                    munew_hbm, snew_hbm,
                    idx_v, x_v, row_v, upd_v, w_v,
                    sem_t, sem_d, sem_r, sem_w, table_sh):
    cid = jax.lax.axis_index("c")
    sid = jax.lax.axis_index("s")
    tsl = pl.ds(jnp.minimum(sid * 128, C - 128), 128)
    bsl = pl.ds(sid * SPB, SPB)

    pltpu.sync_copy(j_hbm.at[bsl], idx_v)

    @pl.when(cid == 0)
    def _():
        tcopy = pltpu.async_copy(mu_hbm.at[tsl], table_sh.at[tsl], sem_t)
        dcopy = pltpu.async_copy(data_hbm.at[bsl], x_v, sem_d)
        rcopy = pltpu.async_copy(mu_hbm.at[idx_v], row_v, sem_r)
        wcopy = pltpu.async_copy(winv_hbm.at[idx_v], w_v, sem_w)
        dcopy.wait()
        rcopy.wait()
        wcopy.wait()

        def rbody(r, carry):
            w16 = w_v[r, pl.ds(0, 16)]
            for k in range(D // 16):
                csl = pl.ds(k * 16, 16)
                upd_v[r, csl] = (x_v[r, csl] - row_v[r, csl]) * w16
            return carry

        jax.lax.fori_loop(0, SPB, rbody, 0)
        tcopy.wait()

    @pl.when(cid == 1)
    def _():
        tcopy = pltpu.async_copy(s_hbm.at[tsl], table_sh.at[tsl], sem_t)
        dcopy = pltpu.async_copy(data_hbm.at[bsl], x_v, sem_d)
        rcopy = pltpu.async_copy(mu_hbm.at[idx_v], row_v, sem_r)
        dcopy.wait()
        rcopy.wait()

        def rbody(r, carry):
            for k in range(D // 16):
                csl = pl.ds(k * 16, 16)
                e16 = x_v[r, csl] - row_v[r, csl]
                upd_v[r, csl] = e16 * e16
            return carry

        jax.lax.fori_loop(0, SPB, rbody, 0)
        tcopy.wait()

    plsc.subcore_barrier()
    pltpu.sync_copy(upd_v, table_sh.at[idx_v], add=True)
    plsc.subcore_barrier()

    @pl.when(cid == 0)
    def _():
        pltpu.sync_copy(table_sh.at[tsl], munew_hbm.at[tsl])

    @pl.when(cid == 1)
    def _():
        pltpu.sync_copy(table_sh.at[tsl], snew_hbm.at[tsl])


def _update_call(data, j, winv, mu, S_diag):
    mesh = plsc.VectorSubcoreMesh(core_axis_name="c", subcore_axis_name="s")
    kern = pl.kernel(
        _sc_update_body,
        out_type=[
            jax.ShapeDtypeStruct((C, D), jnp.float32),
            jax.ShapeDtypeStruct((C, D), jnp.float32),
        ],
        mesh=mesh,
        scratch_types=[
            pltpu.VMEM((SPB,), jnp.int32),
            pltpu.VMEM((SPB, D), jnp.float32),
            pltpu.VMEM((SPB, D), jnp.float32),
            pltpu.VMEM((SPB, D), jnp.float32),
            pltpu.VMEM((SPB, D), jnp.float32),
            pltpu.SemaphoreType.DMA,
            pltpu.SemaphoreType.DMA,
            pltpu.SemaphoreType.DMA,
            pltpu.SemaphoreType.DMA,
            pltpu.VMEM_SHARED((C, D), jnp.float32),
        ],
    )
    return kern(data, j, winv, mu, S_diag)


def kernel(data, labels, n, mu, S_diag, cluster_labels):
    labels_col = labels[:, None]
    scores, pred, clusters, j, n_new, winv = _activation_call(
        data, labels_col, n, mu, S_diag, cluster_labels)
    mu_new, S_new = _update_call(data, j, winv, mu, S_diag)
    return (scores, pred, clusters, n_new, mu_new, S_new)
